# baseline (speedup 1.0000x reference)
import functools

import jax
import jax.numpy as jnp
from jax import lax
from jax.experimental import pallas as pl
from jax.experimental.pallas import tpu as pltpu
from jax.experimental.pallas import tpu_sc as plsc

_N = 10000
_E = 320000
_H = 128

_NCORE = 2
_NSUB = 16
_NW = _NCORE * _NSUB

_NP = 10240
_CHUNK = _NP // _NSUB
_B = 112
_NBATCH = 96
_CH = 8
_NSLOT = 3
_EPW = _B * _NBATCH
_EPAD = _EPW * _NW
_DUMMY = _N

_R = 1024
_G = 10


_sc_mesh = plsc.VectorSubcoreMesh(core_axis_name="c", subcore_axis_name="s")


def _make_sc_agg(compute_deg):
    nch = _NBATCH // _CH
    out_type = [jax.ShapeDtypeStruct((_NCORE, _NP, _H), jnp.float32)]
    scratch = [
        pltpu.VMEM_SHARED((_NP, _H), jnp.float32),
        pltpu.VMEM((_CH, _B), jnp.int32),
        pltpu.VMEM((_CH, _B), jnp.int32),
        pltpu.VMEM((_CH, _B), jnp.int32),
        pltpu.VMEM((_CH, _B), jnp.int32),
        pltpu.SemaphoreType.DMA,
    ]
    scratch += [pltpu.VMEM((_B, _H), jnp.float32) for _ in range(_NSLOT)]
    scratch += [pltpu.SemaphoreType.DMA for _ in range(_NSLOT)]
    scratch += [pltpu.SemaphoreType.DMA for _ in range(_NSLOT)]
    if compute_deg:
        out_type.append(jax.ShapeDtypeStruct((_NCORE, _NP), jnp.float32))
        scratch += [
            pltpu.VMEM((_B,), jnp.float32),
            pltpu.SemaphoreType.DMA,
            pltpu.VMEM_SHARED((_NP,), jnp.float32),
        ]

    @functools.partial(pl.kernel, out_type=out_type, mesh=_sc_mesh,
                       scratch_types=scratch)
    def sc_agg(feat_hbm, src_hbm, dst_hbm, zrows_hbm, zvec_hbm, ones_hbm,
               *rest):
        if compute_deg:
            psum_hbm, pdeg_hbm, acc_sh, *rest2 = rest
        else:
            psum_hbm, acc_sh, *rest2 = rest
        srcb = rest2[0:2]
        dstb = rest2[2:4]
        isem = rest2[4]
        rows = rest2[5:5 + _NSLOT]
        gsem = rest2[5 + _NSLOT:5 + 2 * _NSLOT]
        tsem = rest2[5 + 2 * _NSLOT:5 + 3 * _NSLOT]
        if compute_deg:
            ones_v, dsem, deg_sh = rest2[5 + 3 * _NSLOT:]

        c = lax.axis_index("c")
        s = lax.axis_index("s")
        wid = s * _NCORE + c
        row0 = s * _CHUNK
        cbase = wid * _NBATCH

        def prefetch(ch, sync=False):
            p = ch % 2
            if sync:
                pltpu.sync_copy(src_hbm.at[pl.ds(cbase + ch * _CH, _CH)],
                                srcb[p])
                pltpu.sync_copy(dst_hbm.at[pl.ds(cbase + ch * _CH, _CH)],
                                dstb[p])
            else:
                pltpu.async_copy(src_hbm.at[pl.ds(cbase + ch * _CH, _CH)],
                                 srcb[p], isem)
                pltpu.async_copy(dst_hbm.at[pl.ds(cbase + ch * _CH, _CH)],
                                 dstb[p], isem)

        def prefetch_wait(ch):
            p = ch % 2
            pltpu.make_async_copy(src_hbm.at[pl.ds(cbase + ch * _CH, _CH)],
                                  srcb[p], isem).wait()
            pltpu.make_async_copy(dst_hbm.at[pl.ds(cbase + ch * _CH, _CH)],
                                  dstb[p], isem).wait()

        prefetch(0, sync=True)
        prefetch(1)
        pltpu.sync_copy(zrows_hbm.at[pl.ds(row0, _CHUNK)],
                        acc_sh.at[pl.ds(row0, _CHUNK)])
        if compute_deg:
            pltpu.sync_copy(zvec_hbm.at[pl.ds(row0, _CHUNK)],
                            deg_sh.at[pl.ds(row0, _CHUNK)])
            pltpu.sync_copy(ones_hbm, ones_v)
        plsc.subcore_barrier()

        def srow(j):
            return srcb[(j // _CH) % 2].at[j % _CH]

        def drow(j):
            return dstb[(j // _CH) % 2].at[j % _CH]

        def gather(j):
            p = j % _NSLOT
            pltpu.async_copy(feat_hbm.at[srow(j)], rows[p], gsem[p])

        def gather_wait(j):
            p = j % _NSLOT
            pltpu.make_async_copy(feat_hbm.at[srow(j)], rows[p],
                                  gsem[p]).wait()

        def scatter(j):
            p = j % _NSLOT
            pltpu.async_copy(rows[p], acc_sh.at[drow(j)], tsem[p], add=True)
            if compute_deg:
                pltpu.async_copy(ones_v, deg_sh.at[drow(j)], dsem, add=True)

        def scatter_wait(j):
            p = j % _NSLOT
            pltpu.make_async_copy(rows[p], acc_sh.at[drow(j)],
                                  tsem[p]).wait()

        def deg_wait():
            pltpu.make_async_copy(ones_v, deg_sh.at[dstb[0].at[0]],
                                  dsem).wait()

        gather(0)
        gather(1)
        for j in range(_NBATCH):
            ch, i = divmod(j, _CH)
            gather_wait(j)
            scatter(j)
            if j >= 1:
                scatter_wait(j - 1)
                if compute_deg:
                    deg_wait()
            if i == 1 and 1 <= ch < nch - 1:
                prefetch(ch + 1)
            if i == _CH - 2 and ch < nch - 1:
                prefetch_wait(ch + 1)
            if j + 2 < _NBATCH:
                gather(j + 2)
        scatter_wait(_NBATCH - 1)
        if compute_deg:
            deg_wait()
        plsc.subcore_barrier()

        pltpu.sync_copy(acc_sh.at[pl.ds(row0, _CHUNK)],
                        psum_hbm.at[c, pl.ds(row0, _CHUNK)])
        if compute_deg:
            pltpu.sync_copy(deg_sh.at[pl.ds(row0, _CHUNK)],
                            pdeg_hbm.at[c, pl.ds(row0, _CHUNK)])

    return sc_agg


_sc_agg_deg = _make_sc_agg(True)
_sc_agg_nodeg = _make_sc_agg(False)


def _layer_block(p_ref, dg_ref, x_ref, wl_ref, wr_ref, b_ref, g_ref, be_ref):
    psum = p_ref[0] + p_ref[1]
    deg = dg_ref[0] + dg_ref[1]
    mean = psum / jnp.maximum(deg, 1.0)
    h = jnp.dot(mean, wl_ref[...], preferred_element_type=jnp.float32)
    h = h + jnp.dot(x_ref[...], wr_ref[...], preferred_element_type=jnp.float32)
    h = h + b_ref[...]
    mu = jnp.mean(h, axis=-1, keepdims=True)
    var = jnp.mean((h - mu) * (h - mu), axis=-1, keepdims=True)
    h = (h - mu) * lax.rsqrt(var + 1e-5) * g_ref[...] + be_ref[...]
    return jnp.maximum(h, 0.0)


def _dense0_body(p_ref, dg_ref, x_ref, wl_ref, wr_ref, b_ref, g_ref, be_ref,
                 o_ref):
    o_ref[...] = _layer_block(p_ref, dg_ref, x_ref, wl_ref, wr_ref, b_ref,
                              g_ref, be_ref)


def _dense1_body(p_ref, dg_ref, h_ref, wl_ref, wr_ref, b_ref, g_ref, be_ref,
                 cw1_ref, cb1_ref, cw2_ref, cb2_ref, o_ref):
    h1 = _layer_block(p_ref, dg_ref, h_ref, wl_ref, wr_ref, b_ref, g_ref,
                      be_ref)
    t = jnp.dot(h1, cw1_ref[...], preferred_element_type=jnp.float32)
    t = jnp.maximum(t + cb1_ref[...], 0.0)
    o_ref[...] = (jnp.dot(t, cw2_ref[...], preferred_element_type=jnp.float32)
                  + cb2_ref[...])


_full = pl.BlockSpec((_H, _H), lambda i: (0, 0))
_brow = pl.BlockSpec((1, _H), lambda i: (0, 0))
_pspec = pl.BlockSpec((_NCORE, _R, _H), lambda i: (0, i, 0))
_dgspec = pl.BlockSpec((_NCORE, _R, 1), lambda i: (0, i, 0))
_rowspec = pl.BlockSpec((_R, _H), lambda i: (i, 0))

_dense0 = pl.pallas_call(
    _dense0_body,
    grid=(_G,),
    in_specs=[_pspec, _dgspec, _rowspec, _full, _full, _brow, _brow, _brow],
    out_specs=_rowspec,
    out_shape=jax.ShapeDtypeStruct((_N, _H), jnp.float32),
)

_dense1 = pl.pallas_call(
    _dense1_body,
    grid=(_G,),
    in_specs=[_pspec, _dgspec, _rowspec, _full, _full, _brow, _brow, _brow,
              _full, _brow, pl.BlockSpec((_H, 1), lambda i: (0, 0)),
              pl.BlockSpec((1, 1), lambda i: (0, 0))],
    out_specs=pl.BlockSpec((_R, 1), lambda i: (i, 0)),
    out_shape=jax.ShapeDtypeStruct((_N, 1), jnp.float32),
)


def kernel(x, edge_index, Wl0, Wr0, b0, g0, be0, Wl1, Wr1, b1, g1, be1,
           cW1, cb1, cW2, cb2):
    npad = _EPAD - _E
    pad_iota = lax.iota(jnp.int32, npad)
    src = jnp.concatenate([edge_index[0], pad_iota % _N])
    dst = jnp.concatenate([edge_index[1], _DUMMY + pad_iota % (_NP - _N)])
    src = src.reshape(_NW * _NBATCH, _B)
    dst = dst.reshape(_NW * _NBATCH, _B)
    zrows = jnp.zeros((_NP, _H), jnp.float32)
    zvec = jnp.zeros((_NP,), jnp.float32)
    ones = jnp.ones((_B,), jnp.float32)

    p0, dg = _sc_agg_deg(x, src, dst, zrows, zvec, ones)
    dg3 = dg.reshape(_NCORE, _NP, 1)
    h0 = _dense0(p0, dg3, x, Wl0, Wr0, b0.reshape(1, _H), g0.reshape(1, _H),
                 be0.reshape(1, _H))
    (p1,) = _sc_agg_nodeg(h0, src, dst, zrows, zvec, ones)
    out = _dense1(p1, dg3, h0, Wl1, Wr1, b1.reshape(1, _H),
                  g1.reshape(1, _H), be1.reshape(1, _H),
                  cW1, cb1.reshape(1, _H), cW2, cb2.reshape(1, 1))
    return out


# speedup vs baseline: 1.0310x; 1.0310x over previous
import functools

import jax
import jax.numpy as jnp
from jax import lax
from jax.experimental import pallas as pl
from jax.experimental.pallas import tpu as pltpu
from jax.experimental.pallas import tpu_sc as plsc

_N = 10000
_E = 320000
_H = 128

_NCORE = 2
_NSUB = 16
_NW = _NCORE * _NSUB

_NP = 10240
_CHUNK = _NP // _NSUB
_B = 80
_NBATCH = 128
_CH = 32
_NSLOT = 3
_EPW = _B * _NBATCH
_EPAD = _EPW * _NW
_DUMMY = _N

_R = 1024
_G = 10


_sc_mesh = plsc.VectorSubcoreMesh(core_axis_name="c", subcore_axis_name="s")


def _make_sc_agg(compute_deg):
    nch = _NBATCH // _CH
    out_type = [jax.ShapeDtypeStruct((_NCORE, _NP, _H), jnp.float32)]
    scratch = [
        pltpu.VMEM_SHARED((_NP, _H), jnp.float32),
        pltpu.VMEM((_CH, _B), jnp.int32),
        pltpu.VMEM((_CH, _B), jnp.int32),
        pltpu.VMEM((_CH, _B), jnp.int32),
        pltpu.VMEM((_CH, _B), jnp.int32),
        pltpu.SemaphoreType.DMA,
    ]
    scratch += [pltpu.VMEM((_B, _H), jnp.float32) for _ in range(_NSLOT)]
    scratch += [pltpu.SemaphoreType.DMA for _ in range(_NSLOT)]
    scratch += [pltpu.SemaphoreType.DMA for _ in range(_NSLOT)]
    if compute_deg:
        out_type.append(jax.ShapeDtypeStruct((_NCORE, _NP), jnp.float32))
        scratch += [
            pltpu.VMEM((_B,), jnp.float32),
            pltpu.SemaphoreType.DMA,
            pltpu.VMEM_SHARED((_NP,), jnp.float32),
        ]

    @functools.partial(pl.kernel, out_type=out_type, mesh=_sc_mesh,
                       scratch_types=scratch)
    def sc_agg(feat_hbm, src_hbm, dst_hbm, zrows_hbm, zvec_hbm, ones_hbm,
               *rest):
        if compute_deg:
            psum_hbm, pdeg_hbm, acc_sh, *rest2 = rest
        else:
            psum_hbm, acc_sh, *rest2 = rest
        srcb = rest2[0:2]
        dstb = rest2[2:4]
        isem = rest2[4]
        rows = rest2[5:5 + _NSLOT]
        gsem = rest2[5 + _NSLOT:5 + 2 * _NSLOT]
        tsem = rest2[5 + 2 * _NSLOT:5 + 3 * _NSLOT]
        if compute_deg:
            ones_v, dsem, deg_sh = rest2[5 + 3 * _NSLOT:]

        c = lax.axis_index("c")
        s = lax.axis_index("s")
        wid = s * _NCORE + c
        row0 = s * _CHUNK
        cbase = wid * _NBATCH

        def prefetch(ch, sync=False):
            p = ch % 2
            if sync:
                pltpu.sync_copy(src_hbm.at[pl.ds(cbase + ch * _CH, _CH)],
                                srcb[p])
                pltpu.sync_copy(dst_hbm.at[pl.ds(cbase + ch * _CH, _CH)],
                                dstb[p])
            else:
                pltpu.async_copy(src_hbm.at[pl.ds(cbase + ch * _CH, _CH)],
                                 srcb[p], isem)
                pltpu.async_copy(dst_hbm.at[pl.ds(cbase + ch * _CH, _CH)],
                                 dstb[p], isem)

        def prefetch_wait(ch):
            p = ch % 2
            pltpu.make_async_copy(src_hbm.at[pl.ds(cbase + ch * _CH, _CH)],
                                  srcb[p], isem).wait()
            pltpu.make_async_copy(dst_hbm.at[pl.ds(cbase + ch * _CH, _CH)],
                                  dstb[p], isem).wait()

        prefetch(0, sync=True)
        prefetch(1)
        pltpu.sync_copy(zrows_hbm.at[pl.ds(row0, _CHUNK)],
                        acc_sh.at[pl.ds(row0, _CHUNK)])
        if compute_deg:
            pltpu.sync_copy(zvec_hbm.at[pl.ds(row0, _CHUNK)],
                            deg_sh.at[pl.ds(row0, _CHUNK)])
            pltpu.sync_copy(ones_hbm, ones_v)
        plsc.subcore_barrier()

        def srow(j):
            return srcb[(j // _CH) % 2].at[j % _CH]

        def drow(j):
            return dstb[(j // _CH) % 2].at[j % _CH]

        def gather(j):
            p = j % _NSLOT
            pltpu.async_copy(feat_hbm.at[srow(j)], rows[p], gsem[p])

        def gather_wait(j):
            p = j % _NSLOT
            pltpu.make_async_copy(feat_hbm.at[srow(j)], rows[p],
                                  gsem[p]).wait()

        def scatter(j):
            p = j % _NSLOT
            pltpu.async_copy(rows[p], acc_sh.at[drow(j)], tsem[p], add=True)
            if compute_deg:
                pltpu.async_copy(ones_v, deg_sh.at[drow(j)], dsem, add=True)

        def scatter_wait(j):
            p = j % _NSLOT
            pltpu.make_async_copy(rows[p], acc_sh.at[drow(j)],
                                  tsem[p]).wait()

        def deg_wait():
            pltpu.make_async_copy(ones_v, deg_sh.at[dstb[0].at[0]],
                                  dsem).wait()

        gather(0)
        gather(1)
        for j in range(_NBATCH):
            ch, i = divmod(j, _CH)
            gather_wait(j)
            scatter(j)
            if j >= 1:
                scatter_wait(j - 1)
                if compute_deg:
                    deg_wait()
            if i == 1 and 1 <= ch < nch - 1:
                prefetch(ch + 1)
            if i == _CH - 2 and ch < nch - 1:
                prefetch_wait(ch + 1)
            if j + 2 < _NBATCH:
                gather(j + 2)
        scatter_wait(_NBATCH - 1)
        if compute_deg:
            deg_wait()
        plsc.subcore_barrier()

        pltpu.sync_copy(acc_sh.at[pl.ds(row0, _CHUNK)],
                        psum_hbm.at[c, pl.ds(row0, _CHUNK)])
        if compute_deg:
            pltpu.sync_copy(deg_sh.at[pl.ds(row0, _CHUNK)],
                            pdeg_hbm.at[c, pl.ds(row0, _CHUNK)])

    return sc_agg


_sc_agg_deg = _make_sc_agg(True)
_sc_agg_nodeg = _make_sc_agg(False)


def _layer_block(p_ref, dg_ref, x_ref, wl_ref, wr_ref, b_ref, g_ref, be_ref):
    psum = p_ref[0] + p_ref[1]
    deg = dg_ref[0] + dg_ref[1]
    mean = psum / jnp.maximum(deg, 1.0)
    h = jnp.dot(mean, wl_ref[...], preferred_element_type=jnp.float32)
    h = h + jnp.dot(x_ref[...], wr_ref[...], preferred_element_type=jnp.float32)
    h = h + b_ref[...]
    mu = jnp.mean(h, axis=-1, keepdims=True)
    var = jnp.mean((h - mu) * (h - mu), axis=-1, keepdims=True)
    h = (h - mu) * lax.rsqrt(var + 1e-5) * g_ref[...] + be_ref[...]
    return jnp.maximum(h, 0.0)


def _dense0_body(p_ref, dg_ref, x_ref, wl_ref, wr_ref, b_ref, g_ref, be_ref,
                 o_ref):
    o_ref[...] = _layer_block(p_ref, dg_ref, x_ref, wl_ref, wr_ref, b_ref,
                              g_ref, be_ref)


def _dense1_body(p_ref, dg_ref, h_ref, wl_ref, wr_ref, b_ref, g_ref, be_ref,
                 cw1_ref, cb1_ref, cw2_ref, cb2_ref, o_ref):
    h1 = _layer_block(p_ref, dg_ref, h_ref, wl_ref, wr_ref, b_ref, g_ref,
                      be_ref)
    t = jnp.dot(h1, cw1_ref[...], preferred_element_type=jnp.float32)
    t = jnp.maximum(t + cb1_ref[...], 0.0)
    o_ref[...] = (jnp.dot(t, cw2_ref[...], preferred_element_type=jnp.float32)
                  + cb2_ref[...])


_full = pl.BlockSpec((_H, _H), lambda i: (0, 0))
_brow = pl.BlockSpec((1, _H), lambda i: (0, 0))
_pspec = pl.BlockSpec((_NCORE, _R, _H), lambda i: (0, i, 0))
_dgspec = pl.BlockSpec((_NCORE, _R, 1), lambda i: (0, i, 0))
_rowspec = pl.BlockSpec((_R, _H), lambda i: (i, 0))

_dense0 = pl.pallas_call(
    _dense0_body,
    grid=(_G,),
    in_specs=[_pspec, _dgspec, _rowspec, _full, _full, _brow, _brow, _brow],
    out_specs=_rowspec,
    out_shape=jax.ShapeDtypeStruct((_N, _H), jnp.float32),
)

_dense1 = pl.pallas_call(
    _dense1_body,
    grid=(_G,),
    in_specs=[_pspec, _dgspec, _rowspec, _full, _full, _brow, _brow, _brow,
              _full, _brow, pl.BlockSpec((_H, 1), lambda i: (0, 0)),
              pl.BlockSpec((1, 1), lambda i: (0, 0))],
    out_specs=pl.BlockSpec((_R, 1), lambda i: (i, 0)),
    out_shape=jax.ShapeDtypeStruct((_N, 1), jnp.float32),
)


def kernel(x, edge_index, Wl0, Wr0, b0, g0, be0, Wl1, Wr1, b1, g1, be1,
           cW1, cb1, cW2, cb2):
    npad = _EPAD - _E
    pad_iota = lax.iota(jnp.int32, npad)
    src = jnp.concatenate([edge_index[0], pad_iota % _N])
    dst = jnp.concatenate([edge_index[1], _DUMMY + pad_iota % (_NP - _N)])
    src = src.reshape(_NW * _NBATCH, _B)
    dst = dst.reshape(_NW * _NBATCH, _B)
    zrows = jnp.zeros((_NP, _H), jnp.float32)
    zvec = jnp.zeros((_NP,), jnp.float32)
    ones = jnp.ones((_B,), jnp.float32)

    p0, dg = _sc_agg_deg(x, src, dst, zrows, zvec, ones)
    dg3 = dg.reshape(_NCORE, _NP, 1)
    h0 = _dense0(p0, dg3, x, Wl0, Wr0, b0.reshape(1, _H), g0.reshape(1, _H),
                 be0.reshape(1, _H))
    (p1,) = _sc_agg_nodeg(h0, src, dst, zrows, zvec, ones)
    out = _dense1(p1, dg3, h0, Wl1, Wr1, b1.reshape(1, _H),
                  g1.reshape(1, _H), be1.reshape(1, _H),
                  cW1, cb1.reshape(1, _H), cW2, cb2.reshape(1, 1))
    return out
